# Initial kernel scaffold; baseline (speedup 1.0000x reference)
import jax
import jax.numpy as jnp
from jax.experimental import pallas as pl


def kernel(x, enc_W0, enc_b0, enc_W1, enc_b1, enc_W2, enc_b2, enc_W3, enc_b3, dec_W0, dec_b0, dec_W1, dec_b1, dec_W2, dec_b2, dec_W3, dec_b3, cb0, cb1, cb2, cb3):
    raise NotImplementedError("write your pallas kernel here")


# speedup vs baseline: 3.0790x; 3.0790x over previous
import jax
import jax.numpy as jnp
from jax.experimental import pallas as pl
from jax.experimental.pallas import tpu as pltpu

_BATCH = 4096
_IN_DIM = 768
_E_DIM = 32
_N_CODE = 256
_N_LEVELS = 4
_TB = 512
_LOSS_SCALE = 1.25 / (_N_LEVELS * _BATCH * _E_DIM)


def _dot_t(a, b):
    return jax.lax.dot_general(a, b, (((1,), (1,)), ((), ())),
                               preferred_element_type=jnp.float32)


def _fused_body(x_ref,
                ew0, eb0, ew1, eb1, ew2, eb2, ew3, eb3,
                dw0, db0, dw1, db1, dw2, db2, dw3, db3,
                cb0, cb1, cb2, cb3,
                out_ref, loss_ref, idx_ref, oh_ref, lg_ref):
    h = x_ref[...]
    for w_ref, b_ref, act in ((ew0, eb0, True), (ew1, eb1, True),
                              (ew2, eb2, True), (ew3, eb3, False)):
        h = _dot_t(h, w_ref[...]) + b_ref[...]
        if act:
            h = jnp.maximum(h, 0.0)

    residual = h
    xq_sum = jnp.zeros_like(h)
    loss_sum = jnp.float32(0.0)
    col = jax.lax.broadcasted_iota(jnp.int32, (_TB, _N_CODE), 1)
    for l, cb_ref in enumerate((cb0, cb1, cb2, cb3)):
        cb = cb_ref[...]
        rn = jnp.sum(residual * residual, axis=1, keepdims=True)
        cn = jnp.sum(cb * cb, axis=1)[None, :]
        d = rn + cn - 2.0 * _dot_t(residual, cb)
        dmin = jnp.min(d, axis=1, keepdims=True)
        idx = jnp.min(jnp.where(d == dmin, col, _N_CODE), axis=1)
        oh = (col == idx[:, None]).astype(jnp.float32)
        xq = jax.lax.dot_general(oh, cb, (((1,), (0,)), ((), ())),
                                 preferred_element_type=jnp.float32)
        loss_sum += jnp.sum((xq - residual) ** 2)
        lg_ref[:, l, :] = d
        oh_ref[:, l, :] = oh
        idx_ref[l, :] = idx
        residual = residual - xq
        xq_sum = xq_sum + xq

    h = xq_sum
    for w_ref, b_ref, act in ((dw0, db0, True), (dw1, db1, True),
                              (dw2, db2, True), (dw3, db3, False)):
        h = _dot_t(h, w_ref[...]) + b_ref[...]
        if act:
            h = jnp.maximum(h, 0.0)
    out_ref[...] = h

    step_loss = jnp.reshape(loss_sum * _LOSS_SCALE, (1, 1))
    i = pl.program_id(0)

    @pl.when(i == 0)
    def _init():
        loss_ref[...] = step_loss

    @pl.when(i > 0)
    def _acc():
        loss_ref[...] = loss_ref[...] + step_loss


def kernel(x, enc_W0, enc_b0, enc_W1, enc_b1, enc_W2, enc_b2, enc_W3, enc_b3,
           dec_W0, dec_b0, dec_W1, dec_b1, dec_W2, dec_b2, dec_W3, dec_b3,
           cb0, cb1, cb2, cb3):
    f32 = jnp.float32
    ebs = [b.reshape(1, -1) for b in (enc_b0, enc_b1, enc_b2, enc_b3)]
    dbs = [b.reshape(1, -1) for b in (dec_b0, dec_b1, dec_b2, dec_b3)]
    ews = (enc_W0, enc_W1, enc_W2, enc_W3)
    dws = (dec_W0, dec_W1, dec_W2, dec_W3)
    cbs = (cb0, cb1, cb2, cb3)

    grid = (_BATCH // _TB,)
    full = lambda a: pl.BlockSpec(a.shape, lambda i: (0,) * a.ndim)

    in_specs = [pl.BlockSpec((_TB, _IN_DIM), lambda i: (i, 0))]
    operands = [x]
    for w, b in zip(ews, ebs):
        in_specs += [full(w), full(b)]
        operands += [w, b]
    for w, b in zip(dws, dbs):
        in_specs += [full(w), full(b)]
        operands += [w, b]
    for cb in cbs:
        in_specs.append(full(cb))
        operands.append(cb)

    out_shapes = (
        jax.ShapeDtypeStruct((_BATCH, _IN_DIM), f32),
        jax.ShapeDtypeStruct((1, 1), f32),
        jax.ShapeDtypeStruct((_N_LEVELS, _BATCH), jnp.int32),
        jax.ShapeDtypeStruct((_BATCH, _N_LEVELS, _N_CODE), f32),
        jax.ShapeDtypeStruct((_BATCH, _N_LEVELS, _N_CODE), f32),
    )
    out_specs = (
        pl.BlockSpec((_TB, _IN_DIM), lambda i: (i, 0)),
        pl.BlockSpec((1, 1), lambda i: (0, 0)),
        pl.BlockSpec((_N_LEVELS, _TB), lambda i: (0, i)),
        pl.BlockSpec((_TB, _N_LEVELS, _N_CODE), lambda i: (i, 0, 0)),
        pl.BlockSpec((_TB, _N_LEVELS, _N_CODE), lambda i: (i, 0, 0)),
    )

    out, loss, idx_t, oh, lg = pl.pallas_call(
        _fused_body,
        grid=grid,
        in_specs=in_specs,
        out_specs=out_specs,
        out_shape=out_shapes,
        compiler_params=pltpu.CompilerParams(
            dimension_semantics=("arbitrary",)),
    )(*operands)

    return out, loss[0, 0], idx_t.T, oh, lg


# speedup vs baseline: 3.2942x; 1.0699x over previous
import jax
import jax.numpy as jnp
from jax.experimental import pallas as pl
from jax.experimental.pallas import tpu as pltpu

_BATCH = 4096
_IN_DIM = 768
_E_DIM = 32
_N_CODE = 256
_N_LEVELS = 4
_TB = 512
_LOSS_SCALE = 1.25 / (_N_LEVELS * _BATCH * _E_DIM)


def _dot_t(a, b):
    return jax.lax.dot_general(a, b, (((1,), (1,)), ((), ())),
                               preferred_element_type=jnp.float32)


def _fused_body(x_ref,
                ew0, eb0, ew1, eb1, ew2, eb2, ew3, eb3,
                dw0, db0, dw1, db1, dw2, db2, dw3, db3,
                cb0, cb1, cb2, cb3,
                out_ref, loss_ref, idx_ref, oh_ref, lg_ref):
    h = x_ref[...]
    for w_ref, b_ref, act in ((ew0, eb0, True), (ew1, eb1, True),
                              (ew2, eb2, True), (ew3, eb3, False)):
        h = _dot_t(h, w_ref[...]) + b_ref[...]
        if act:
            h = jnp.maximum(h, 0.0)

    residual = h
    xq_sum = jnp.zeros_like(h)
    loss_sum = jnp.float32(0.0)
    col = jax.lax.broadcasted_iota(
        jnp.int32, (_TB, _N_CODE), 1).astype(jnp.float32)
    for l, cb_ref in enumerate((cb0, cb1, cb2, cb3)):
        cb = cb_ref[...]
        rn = jnp.sum(residual * residual, axis=1, keepdims=True)
        cn = jnp.sum(cb * cb, axis=1)[None, :]
        d = rn + cn - 2.0 * _dot_t(residual, cb)
        dmin = jnp.min(d, axis=1, keepdims=True)
        idx = jnp.min(jnp.where(d == dmin, col, float(_N_CODE)), axis=1,
                      keepdims=True)
        oh = (col == idx).astype(jnp.float32)
        xq = jax.lax.dot_general(oh, cb, (((1,), (0,)), ((), ())),
                                 preferred_element_type=jnp.float32)
        loss_sum += jnp.sum((xq - residual) ** 2)
        lg_ref[:, l, :] = d
        oh_ref[:, l, :] = oh
        idx_ref[:, l:l + 1] = idx.astype(jnp.int32)
        residual = residual - xq
        xq_sum = xq_sum + xq

    h = xq_sum
    for w_ref, b_ref, act in ((dw0, db0, True), (dw1, db1, True),
                              (dw2, db2, True), (dw3, db3, False)):
        h = _dot_t(h, w_ref[...]) + b_ref[...]
        if act:
            h = jnp.maximum(h, 0.0)
    out_ref[...] = h

    step_loss = jnp.reshape(loss_sum * _LOSS_SCALE, (1, 1))
    i = pl.program_id(0)

    @pl.when(i == 0)
    def _init():
        loss_ref[...] = step_loss

    @pl.when(i > 0)
    def _acc():
        loss_ref[...] = loss_ref[...] + step_loss


def kernel(x, enc_W0, enc_b0, enc_W1, enc_b1, enc_W2, enc_b2, enc_W3, enc_b3,
           dec_W0, dec_b0, dec_W1, dec_b1, dec_W2, dec_b2, dec_W3, dec_b3,
           cb0, cb1, cb2, cb3):
    f32 = jnp.float32
    ebs = [b.reshape(1, -1) for b in (enc_b0, enc_b1, enc_b2, enc_b3)]
    dbs = [b.reshape(1, -1) for b in (dec_b0, dec_b1, dec_b2, dec_b3)]
    ews = (enc_W0, enc_W1, enc_W2, enc_W3)
    dws = (dec_W0, dec_W1, dec_W2, dec_W3)
    cbs = (cb0, cb1, cb2, cb3)

    grid = (_BATCH // _TB,)
    full = lambda a: pl.BlockSpec(a.shape, lambda i: (0,) * a.ndim)

    in_specs = [pl.BlockSpec((_TB, _IN_DIM), lambda i: (i, 0))]
    operands = [x]
    for w, b in zip(ews, ebs):
        in_specs += [full(w), full(b)]
        operands += [w, b]
    for w, b in zip(dws, dbs):
        in_specs += [full(w), full(b)]
        operands += [w, b]
    for cb in cbs:
        in_specs.append(full(cb))
        operands.append(cb)

    out_shapes = (
        jax.ShapeDtypeStruct((_BATCH, _IN_DIM), f32),
        jax.ShapeDtypeStruct((1, 1), f32),
        jax.ShapeDtypeStruct((_BATCH, _N_LEVELS), jnp.int32),
        jax.ShapeDtypeStruct((_BATCH, _N_LEVELS, _N_CODE), f32),
        jax.ShapeDtypeStruct((_BATCH, _N_LEVELS, _N_CODE), f32),
    )
    out_specs = (
        pl.BlockSpec((_TB, _IN_DIM), lambda i: (i, 0)),
        pl.BlockSpec((1, 1), lambda i: (0, 0)),
        pl.BlockSpec((_TB, _N_LEVELS), lambda i: (i, 0)),
        pl.BlockSpec((_TB, _N_LEVELS, _N_CODE), lambda i: (i, 0, 0)),
        pl.BlockSpec((_TB, _N_LEVELS, _N_CODE), lambda i: (i, 0, 0)),
    )

    out, loss, idx, oh, lg = pl.pallas_call(
        _fused_body,
        grid=grid,
        in_specs=in_specs,
        out_specs=out_specs,
        out_shape=out_shapes,
        compiler_params=pltpu.CompilerParams(
            dimension_semantics=("arbitrary",)),
    )(*operands)

    return out, loss[0, 0], idx, oh, lg
